# Initial kernel scaffold; baseline (speedup 1.0000x reference)
import jax
import jax.numpy as jnp
from jax.experimental import pallas as pl


def kernel(d_sim, m_sim, W_d, W_m, W_gat, attn_l, attn_r, Wm1, bm1, Wd1, bd1, Wp, bp, edge_index, diseases, mirnas):
    raise NotImplementedError("write your pallas kernel here")


# speedup vs baseline: 45.9088x; 45.9088x over previous
import jax
import jax.numpy as jnp
from jax import lax
from jax.experimental import pallas as pl
from jax.experimental.pallas import tpu as pltpu
from jax.experimental.pallas import tpu_sc as plsc

N = 10000
ND = 5000
NP = 10240
E = 640000
EP = 655360
HID = 64
FH = 32
K = 8
BPAIR = 16384
SLOPE = 0.2
NC, NS, L = 2, 16, 16
NW = NC * NS
RPT = NP // NS
EPT = EP // NS
GEDG = 512
NGRP = EPT // GEDG
EPW = EP // NW
NGRPW = EPW // GEDG
ROWS8 = EP // 128
PADIDX = 10016
f32 = jnp.float32
i32 = jnp.int32

_mesh = plsc.VectorSubcoreMesh(core_axis_name="c", subcore_axis_name="s")
_sc_params = pltpu.CompilerParams(needs_layout_passes=False, use_tc_tiling_on_sc=False)


def _scd_body(dst2, degs, didxA, didxB, deg_t, isemA, isemB):
    cid = lax.axis_index("c")
    sid = lax.axis_index("s")
    wid = sid * NC + cid
    grp0 = wid * (EPW // 128)

    z16 = jnp.zeros((L,), f32)
    ones16 = jnp.ones((L,), f32)

    def zb(i, _):
        deg_t[pl.ds(i * L, L)] = z16
        return 0
    lax.fori_loop(0, NP // L, zb, 0)

    def issue(slot, g):
        db, isem = (didxA, isemA) if slot == 0 else (didxB, isemB)
        pltpu.async_copy(dst2.at[pl.ds(grp0 + g * 4, 4)], db, isem)

    def wait(slot):
        db, isem = (didxA, isemA) if slot == 0 else (didxB, isemB)
        pltpu.make_async_copy(dst2.at[pl.ds(0, 4)], db, isem).wait()

    def consume(slot):
        db = didxA if slot == 0 else didxB

        def cb(q, _):
            for jj in range(8):
                iv = db[q, pl.ds(jj * L, L)]
                plsc.addupdate_scatter(deg_t, [iv], ones16)
            return 0
        lax.fori_loop(0, 4, cb, 0)

    issue(0, 0)
    issue(1, 1)

    def pair(p, _):
        g = 2 * p
        wait(0)
        consume(0)

        @pl.when(p < NGRPW // 2 - 1)
        def _():
            issue(0, g + 2)

        wait(1)
        consume(1)

        @pl.when(p < NGRPW // 2 - 1)
        def _():
            issue(1, g + 3)

        return 0
    lax.fori_loop(0, NGRPW // 2, pair, 0)

    pltpu.sync_copy(deg_t, degs.at[wid])


def _scd(dst2):
    return pl.kernel(
        _scd_body,
        out_type=[jax.ShapeDtypeStruct((NW, NP), f32)],
        mesh=_mesh,
        compiler_params=_sc_params,
        scratch_types=[
            pltpu.VMEM((4, 128), i32),
            pltpu.VMEM((4, 128), i32),
            pltpu.VMEM((NP,), f32),
            pltpu.SemaphoreType.DMA,
            pltpu.SemaphoreType.DMA,
        ],
    )(dst2)


def _tc1_body(d_ref, m_ref, wd_ref, wm_ref, degs_ref,
              feats_ref, x0h_ref, norm_ref):
    zd = jnp.dot(d_ref[...], wd_ref[...], preferred_element_type=f32)
    zm = jnp.dot(m_ref[...], wm_ref[...], preferred_element_type=f32)
    feats_ref[pl.ds(0, ND), :] = zd
    feats_ref[pl.ds(ND, ND), :] = zm
    feats_ref[pl.ds(N, NP - N), :] = jnp.zeros((NP - N, HID), f32)
    f = feats_ref[...]
    x0h_ref[0] = 0.5 * f[:, :FH]
    x0h_ref[1] = 0.5 * f[:, FH:]
    deg = jnp.clip(jnp.sum(degs_ref[...], axis=0), 1.0, None)
    norm_ref[...] = lax.rsqrt(deg)


def _tc1(d_sim, m_sim, W_d, W_m, degs):
    return pl.pallas_call(
        _tc1_body,
        out_shape=[
            jax.ShapeDtypeStruct((NP, HID), f32),
            jax.ShapeDtypeStruct((2, NP, FH), f32),
            jax.ShapeDtypeStruct((NP,), f32),
        ],
    )(d_sim, m_sim, W_d, W_m, degs)


def _tc2_body(yh_ref, wg_ref, al_ref, ar_ref, h2_ref, el_ref, er_ref, c_ref):
    X = jnp.concatenate([yh_ref[0], yh_ref[1]], axis=1)
    h = jnp.dot(X, wg_ref[...], preferred_element_type=f32)
    el = jnp.dot(h, al_ref[...], preferred_element_type=f32)
    er = jnp.dot(h, ar_ref[...], preferred_element_type=f32)
    h2_ref[0] = h[:, :FH]
    h2_ref[1] = h[:, FH:]
    el_ref[...] = el
    er_ref[...] = er
    t = jnp.max(el) + jnp.max(er)
    c = jnp.where(t > 0, t, SLOPE * t)
    c_ref[...] = jnp.full((128,), c, f32)


def _tc2(yh, W_gat, attn_l, attn_r):
    return pl.pallas_call(
        _tc2_body,
        out_shape=[
            jax.ShapeDtypeStruct((2, NP, FH), f32),
            jax.ShapeDtypeStruct((NP,), f32),
            jax.ShapeDtypeStruct((NP,), f32),
            jax.ShapeDtypeStruct((128,), f32),
        ],
    )(yh, W_gat, attn_l, attn_r)


def _tc3_body(numer_ref, dens_ref, feats_ref, wm1_ref, bm1_ref, wd1_ref,
              bd1_ref, wp_ref, bp_ref, s1_ref, s2_ref):
    den = jnp.clip(jnp.sum(dens_ref[...], axis=0), 1e-9, None)
    gat = jnp.concatenate([numer_ref[0], numer_ref[1]], axis=1) / den[:, None]
    m = jnp.max(gat, axis=-1, keepdims=True)
    feat0 = gat - (m + jnp.log(jnp.sum(jnp.exp(gat - m), axis=-1, keepdims=True)))
    f = feats_ref[...]
    wd1 = wd1_ref[...]
    wm1 = wm1_ref[...]
    a_d = (jnp.dot(feat0[:ND], wd1[:HID], preferred_element_type=f32)
           + jnp.dot(f[:ND], wd1[HID:], preferred_element_type=f32)
           + bd1_ref[...])
    a_m = (jnp.dot(feat0[ND:N], wm1[:HID], preferred_element_type=f32)
           + jnp.dot(f[ND:N], wm1[HID:], preferred_element_type=f32)
           + bm1_ref[...])
    h_d = jnp.where(a_d > 0, a_d, jnp.exp(a_d) - 1.0)
    h_m = jnp.where(a_m > 0, a_m, jnp.exp(a_m) - 1.0)
    wp1 = wp_ref[...][:HID, 0]
    wp2 = wp_ref[...][HID:, 0]
    bp = bp_ref[...]
    s1_ref[pl.ds(0, ND)] = jnp.dot(h_d, wp1, preferred_element_type=f32) + bp
    s1_ref[pl.ds(ND, ND)] = jnp.dot(h_m, wp1, preferred_element_type=f32) + bp
    s1_ref[pl.ds(N, NP - N)] = jnp.zeros((NP - N,), f32)
    s2_ref[pl.ds(0, ND)] = jnp.dot(h_d, wp2, preferred_element_type=f32)
    s2_ref[pl.ds(ND, ND)] = jnp.dot(h_m, wp2, preferred_element_type=f32)
    s2_ref[pl.ds(N, NP - N)] = jnp.zeros((NP - N,), f32)


def _tc3(numer, dens, featsP, Wm1, bm1, Wd1, bd1, Wp, bp):
    return pl.pallas_call(
        _tc3_body,
        out_shape=[
            jax.ShapeDtypeStruct((NP,), f32),
            jax.ShapeDtypeStruct((NP,), f32),
        ],
    )(numer, dens, featsP, Wm1, bm1, Wd1, bd1, Wp, bp)


def _sca_body(x0h, src2, dst2, normP, yh,
              u_sp, s_sp,
              sidxA, sidxB, didxA, didxB, gbufA, gbufB, zgbuf,
              xsl, ysl, normsv,
              isemA, isemB, gsemA, gsemB, ssemA, ssemB):
    cid = lax.axis_index("c")
    sid = lax.axis_index("s")
    row0 = sid * RPT
    grp0 = sid * (EPT // 128)

    z16 = jnp.zeros((L,), f32)

    def zero2d(ref, rows):
        def b(i, _):
            ref[i, pl.ds(0, L)] = z16
            ref[i, pl.ds(L, L)] = z16
            return 0
        lax.fori_loop(0, rows, b, 0)

    zero2d(ysl, RPT)
    zero2d(zgbuf, 32)
    pltpu.sync_copy(normP.at[pl.ds(row0, RPT)], normsv)

    def issue_idx(slot, g):
        sb, db, isem = ((sidxA, didxA, isemA) if slot == 0
                        else (sidxB, didxB, isemB))
        pltpu.async_copy(src2.at[pl.ds(grp0 + g * 4, 4)], sb, isem)
        pltpu.async_copy(dst2.at[pl.ds(grp0 + g * 4, 4)], db, isem)

    def wait_idx(slot):
        sb, db, isem = ((sidxA, didxA, isemA) if slot == 0
                        else (sidxB, didxB, isemB))
        pltpu.make_async_copy(src2.at[pl.ds(0, 4)], sb, isem).wait()
        pltpu.make_async_copy(dst2.at[pl.ds(0, 4)], db, isem).wait()

    def gathers(slot):
        sb, gb, gsem = ((sidxA, gbufA, gsemA) if slot == 0
                        else (sidxB, gbufB, gsemB))
        for j in range(4):
            pltpu.async_copy(u_sp.at[sb.at[j]],
                             gb.at[pl.ds(j * 128, 128)], gsem)

    def drain_gathers(slot):
        sb, gb, gsem = ((sidxA, gbufA, gsemA) if slot == 0
                        else (sidxB, gbufB, gsemB))
        for j in range(4):
            pltpu.make_async_copy(u_sp.at[sb.at[j]],
                                  gb.at[pl.ds(j * 128, 128)], gsem).wait()

    def scatters(slot):
        db, gb, ssem = ((didxA, gbufA, ssemA) if slot == 0
                        else (didxB, gbufB, ssemB))
        for j in range(4):
            pltpu.async_copy(gb.at[pl.ds(j * 128, 128)],
                             s_sp.at[db.at[j]], ssem, add=True)

    def drain_scatters(slot):
        db, gb, ssem = ((didxA, gbufA, ssemA) if slot == 0
                        else (didxB, gbufB, ssemB))
        for j in range(4):
            pltpu.make_async_copy(gb.at[pl.ds(j * 128, 128)],
                                  s_sp.at[db.at[j]], ssem).wait()

    def edge_pass():
        issue_idx(0, 0)
        issue_idx(1, 1)
        wait_idx(0)
        gathers(0)

        def pair(p, _):
            g = 2 * p
            wait_idx(1)
            drain_gathers(0)
            scatters(0)

            @pl.when(p < NGRP // 2 - 1)
            def _():
                issue_idx(0, g + 2)

            gathers(1)
            drain_gathers(1)
            scatters(1)

            @pl.when(p < NGRP // 2 - 1)
            def _():
                issue_idx(1, g + 3)

            drain_scatters(0)

            @pl.when(p < NGRP // 2 - 1)
            def _():
                wait_idx(0)
                gathers(0)

            drain_scatters(1)
            return 0
        lax.fori_loop(0, NGRP // 2, pair, 0)

    for t in range(K + 1):
        first = t == 0
        last = t == K
        if first:
            pltpu.sync_copy(x0h.at[cid, pl.ds(row0, RPT)], xsl)
        else:
            pltpu.sync_copy(s_sp.at[pl.ds(row0, RPT)], xsl)
        if not last:
            for zi in range(RPT // 32):
                pltpu.sync_copy(zgbuf, s_sp.at[pl.ds(row0 + zi * 32, 32)])

        def rowb(r, _, first=first, last=last):
            nv = plsc.load_gather(normsv, [jnp.full((L,), r, i32)])
            for half in range(2):
                v = xsl[r, pl.ds(half * L, L)]
                if not first:
                    v = v * nv
                yv = ysl[r, pl.ds(half * L, L)] + v
                ysl[r, pl.ds(half * L, L)] = yv
                if last:
                    xsl[r, pl.ds(half * L, L)] = yv * (1.0 / (K + 1))
                else:
                    xsl[r, pl.ds(half * L, L)] = v * nv
            return 0
        lax.fori_loop(0, RPT, rowb, 0)

        if last:
            pltpu.sync_copy(xsl, yh.at[cid, pl.ds(row0, RPT)])
        else:
            pltpu.sync_copy(xsl, u_sp.at[pl.ds(row0, RPT)])
            plsc.subcore_barrier()
            edge_pass()
            plsc.subcore_barrier()


def _sca(x0h, src2, dst2, normP):
    return pl.kernel(
        _sca_body,
        out_type=[jax.ShapeDtypeStruct((2, NP, FH), f32)],
        mesh=_mesh,
        compiler_params=_sc_params,
        scratch_types=[
            pltpu.VMEM_SHARED((NP, FH), f32),
            pltpu.VMEM_SHARED((NP, FH), f32),
            pltpu.VMEM((4, 128), i32),
            pltpu.VMEM((4, 128), i32),
            pltpu.VMEM((4, 128), i32),
            pltpu.VMEM((4, 128), i32),
            pltpu.VMEM((GEDG, FH), f32),
            pltpu.VMEM((GEDG, FH), f32),
            pltpu.VMEM((32, FH), f32),
            pltpu.VMEM((RPT, FH), f32),
            pltpu.VMEM((RPT, FH), f32),
            pltpu.VMEM((RPT,), f32),
            pltpu.SemaphoreType.DMA,
            pltpu.SemaphoreType.DMA,
            pltpu.SemaphoreType.DMA,
            pltpu.SemaphoreType.DMA,
            pltpu.SemaphoreType.DMA,
            pltpu.SemaphoreType.DMA,
        ],
    )(x0h, src2, dst2, normP)


def _scb_body(h2, el, er, crow, src2, dst2, numer, dens,
              h_sp, n_sp,
              sidxA, sidxB, didxA, didxB, gbufA, gbufB, zgbuf,
              elt, ert, dent, eebuf, cbuf,
              isemA, isemB, gsemA, gsemB, ssemA, ssemB):
    cid = lax.axis_index("c")
    sid = lax.axis_index("s")
    row0 = sid * RPT
    grp0 = sid * (EPT // 128)

    z16 = jnp.zeros((L,), f32)

    def zero1d(ref, n16):
        def b(i, _):
            ref[pl.ds(i * L, L)] = z16
            return 0
        lax.fori_loop(0, n16, b, 0)

    def zero2d(ref, rows):
        def b(i, _):
            ref[i, pl.ds(0, L)] = z16
            ref[i, pl.ds(L, L)] = z16
            return 0
        lax.fori_loop(0, rows, b, 0)

    zero1d(dent, NP // L)
    zero2d(zgbuf, 32)

    pltpu.sync_copy(h2.at[cid, pl.ds(row0, RPT)], h_sp.at[pl.ds(row0, RPT)])
    for zi in range(RPT // 32):
        pltpu.sync_copy(zgbuf, n_sp.at[pl.ds(row0 + zi * 32, 32)])
    pltpu.sync_copy(el, elt)
    pltpu.sync_copy(er, ert)
    pltpu.sync_copy(crow.at[pl.ds(0, L)], cbuf)
    cv = cbuf[pl.ds(0, L)]
    plsc.subcore_barrier()

    def issue_idx(slot, g):
        sb, db, isem = ((sidxA, didxA, isemA) if slot == 0
                        else (sidxB, didxB, isemB))
        pltpu.async_copy(src2.at[pl.ds(grp0 + g * 4, 4)], sb, isem)
        pltpu.async_copy(dst2.at[pl.ds(grp0 + g * 4, 4)], db, isem)

    def wait_idx(slot):
        sb, db, isem = ((sidxA, didxA, isemA) if slot == 0
                        else (sidxB, didxB, isemB))
        pltpu.make_async_copy(src2.at[pl.ds(0, 4)], sb, isem).wait()
        pltpu.make_async_copy(dst2.at[pl.ds(0, 4)], db, isem).wait()

    def gathers(slot):
        sb, gb, gsem = ((sidxA, gbufA, gsemA) if slot == 0
                        else (sidxB, gbufB, gsemB))
        for j in range(4):
            pltpu.async_copy(h_sp.at[sb.at[j]],
                             gb.at[pl.ds(j * 128, 128)], gsem)

    def drain_gathers(slot):
        sb, gb, gsem = ((sidxA, gbufA, gsemA) if slot == 0
                        else (sidxB, gbufB, gsemB))
        for j in range(4):
            pltpu.make_async_copy(h_sp.at[sb.at[j]],
                                  gb.at[pl.ds(j * 128, 128)], gsem).wait()

    def scatters(slot):
        db, gb, ssem = ((didxA, gbufA, ssemA) if slot == 0
                        else (didxB, gbufB, ssemB))
        for j in range(4):
            pltpu.async_copy(gb.at[pl.ds(j * 128, 128)],
                             n_sp.at[db.at[j]], ssem, add=True)

    def drain_scatters(slot):
        db, gb, ssem = ((didxA, gbufA, ssemA) if slot == 0
                        else (didxB, gbufB, ssemB))
        for j in range(4):
            pltpu.make_async_copy(gb.at[pl.ds(j * 128, 128)],
                                  n_sp.at[db.at[j]], ssem).wait()

    def edge_compute(slot):
        sb, db, gb = ((sidxA, didxA, gbufA) if slot == 0
                      else (sidxB, didxB, gbufB))

        def cb(q, _):
            for jj in range(8):
                svv = sb[q, pl.ds(jj * L, L)]
                dvv = db[q, pl.ds(jj * L, L)]
                ev = (plsc.load_gather(elt, [svv])
                      + plsc.load_gather(ert, [dvv]))
                ev = jnp.where(ev > 0, ev, SLOPE * ev)
                eev = jnp.exp(ev - cv)
                eebuf[pl.ds(q * 128 + jj * L, L)] = eev
                plsc.addupdate_scatter(dent, [dvv], eev)
            return 0
        lax.fori_loop(0, 4, cb, 0)

        def rb(rr, _):
            for u in range(8):
                eev = plsc.load_gather(eebuf, [jnp.full((L,), rr * 8 + u, i32)])
                gb[rr * 8 + u, pl.ds(0, L)] = gb[rr * 8 + u, pl.ds(0, L)] * eev
                gb[rr * 8 + u, pl.ds(L, L)] = gb[rr * 8 + u, pl.ds(L, L)] * eev
            return 0
        lax.fori_loop(0, GEDG // 8, rb, 0)

    issue_idx(0, 0)
    issue_idx(1, 1)
    wait_idx(0)
    gathers(0)

    def pair(p, _):
        g = 2 * p
        wait_idx(1)
        drain_gathers(0)
        edge_compute(0)
        scatters(0)

        @pl.when(p < NGRP // 2 - 1)
        def _():
            issue_idx(0, g + 2)

        gathers(1)
        drain_gathers(1)
        edge_compute(1)
        scatters(1)

        @pl.when(p < NGRP // 2 - 1)
        def _():
            issue_idx(1, g + 3)

        drain_scatters(0)

        @pl.when(p < NGRP // 2 - 1)
        def _():
            wait_idx(0)
            gathers(0)

        drain_scatters(1)
        return 0
    lax.fori_loop(0, NGRP // 2, pair, 0)

    plsc.subcore_barrier()
    pltpu.sync_copy(n_sp.at[pl.ds(row0, RPT)], numer.at[cid, pl.ds(row0, RPT)])
    pltpu.sync_copy(dent, dens.at[sid])


def _scb(h2, el, er, crow, src2, dst2):
    return pl.kernel(
        _scb_body,
        out_type=[
            jax.ShapeDtypeStruct((2, NP, FH), f32),
            jax.ShapeDtypeStruct((NS, NP), f32),
        ],
        mesh=_mesh,
        compiler_params=_sc_params,
        scratch_types=[
            pltpu.VMEM_SHARED((NP, FH), f32),
            pltpu.VMEM_SHARED((NP, FH), f32),
            pltpu.VMEM((4, 128), i32),
            pltpu.VMEM((4, 128), i32),
            pltpu.VMEM((4, 128), i32),
            pltpu.VMEM((4, 128), i32),
            pltpu.VMEM((GEDG, FH), f32),
            pltpu.VMEM((GEDG, FH), f32),
            pltpu.VMEM((32, FH), f32),
            pltpu.VMEM((NP,), f32),
            pltpu.VMEM((NP,), f32),
            pltpu.VMEM((NP,), f32),
            pltpu.VMEM((GEDG,), f32),
            pltpu.VMEM((L,), f32),
            pltpu.SemaphoreType.DMA,
            pltpu.SemaphoreType.DMA,
            pltpu.SemaphoreType.DMA,
            pltpu.SemaphoreType.DMA,
            pltpu.SemaphoreType.DMA,
            pltpu.SemaphoreType.DMA,
        ],
    )(h2, el, er, crow, src2, dst2)


def _scc_body(s1h, s2h, dis, mir, out, s1t, s2t, dxt, mxt, obuf):
    cid = lax.axis_index("c")
    sid = lax.axis_index("s")
    wid = sid * NC + cid
    npt = BPAIR // NW
    base = wid * npt
    pltpu.sync_copy(s1h, s1t)
    pltpu.sync_copy(s2h, s2t)
    pltpu.sync_copy(dis.at[pl.ds(base, npt)], dxt)
    pltpu.sync_copy(mir.at[pl.ds(base, npt)], mxt)

    def b(j, _):
        iv = dxt[pl.ds(j * L, L)]
        jv = mxt[pl.ds(j * L, L)]
        a = plsc.load_gather(s1t, [iv]) + plsc.load_gather(s2t, [jv])
        obuf[pl.ds(j * L, L)] = 1.0 / (1.0 + jnp.exp(-a))
        return 0
    lax.fori_loop(0, npt // L, b, 0)
    pltpu.sync_copy(obuf, out.at[pl.ds(base, npt)])


def _scc(s1P, s2P, diseases, mirnas):
    npt = BPAIR // NW
    return pl.kernel(
        _scc_body,
        out_type=[jax.ShapeDtypeStruct((BPAIR,), f32)],
        mesh=_mesh,
        compiler_params=_sc_params,
        scratch_types=[
            pltpu.VMEM((NP,), f32),
            pltpu.VMEM((NP,), f32),
            pltpu.VMEM((npt,), i32),
            pltpu.VMEM((npt,), i32),
            pltpu.VMEM((npt,), f32),
        ],
    )(s1P, s2P, diseases, mirnas)


def kernel(d_sim, m_sim, W_d, W_m, W_gat, attn_l, attn_r, Wm1, bm1, Wd1, bd1,
           Wp, bp, edge_index, diseases, mirnas):
    pad = jnp.full((EP - E,), PADIDX, i32)
    src2 = jnp.concatenate([edge_index[0].astype(i32), pad]).reshape(ROWS8, 128)
    dst2 = jnp.concatenate([edge_index[1].astype(i32), pad]).reshape(ROWS8, 128)

    (degs,) = _scd(dst2)
    featsP, x0h, normP = _tc1(d_sim, m_sim, W_d, W_m, degs)
    (yh,) = _sca(x0h, src2, dst2, normP)
    h2, el, er, crow = _tc2(yh, W_gat, attn_l, attn_r)
    numer, dens = _scb(h2, el, er, crow, src2, dst2)
    s1P, s2P = _tc3(numer, dens, featsP, Wm1, bm1, Wd1, bd1, Wp, bp)
    (score,) = _scc(s1P, s2P, diseases.astype(i32), mirnas.astype(i32))
    return score.reshape(BPAIR, 1)


# speedup vs baseline: 50.6942x; 1.1042x over previous
import jax
import jax.numpy as jnp
from jax import lax
from jax.experimental import pallas as pl
from jax.experimental.pallas import tpu as pltpu
from jax.experimental.pallas import tpu_sc as plsc

N = 10000
ND = 5000
NP = 10240
E = 640000
EP = 655360
HID = 64
FH = 32
K = 8
BPAIR = 16384
SLOPE = 0.2
NC, NS, L = 2, 16, 16
NW = NC * NS
RPT = NP // NS
EPT = EP // NS
GEDG = 512
NGRP = EPT // GEDG
EPW = EP // NW
NGRPW = EPW // GEDG
ROWS8 = EP // 128
PADIDX = 10016
f32 = jnp.float32
i32 = jnp.int32

_mesh = plsc.VectorSubcoreMesh(core_axis_name="c", subcore_axis_name="s")
_sc_params = pltpu.CompilerParams(needs_layout_passes=False, use_tc_tiling_on_sc=False)


def _scd_body(dst2, degs, didxA, didxB, deg_t, isemA, isemB):
    cid = lax.axis_index("c")
    sid = lax.axis_index("s")
    wid = sid * NC + cid
    grp0 = wid * (EPW // 128)

    z16 = jnp.zeros((L,), f32)
    ones16 = jnp.ones((L,), f32)

    def zb(i, _):
        deg_t[pl.ds(i * L, L)] = z16
        return 0
    lax.fori_loop(0, NP // L, zb, 0)

    def issue(slot, g):
        db, isem = (didxA, isemA) if slot == 0 else (didxB, isemB)
        pltpu.async_copy(dst2.at[pl.ds(grp0 + g * 4, 4)], db, isem)

    def wait(slot):
        db, isem = (didxA, isemA) if slot == 0 else (didxB, isemB)
        pltpu.make_async_copy(dst2.at[pl.ds(0, 4)], db, isem).wait()

    def consume(slot):
        db = didxA if slot == 0 else didxB

        def cb(q, _):
            for jj in range(8):
                iv = db[q, pl.ds(jj * L, L)]
                plsc.addupdate_scatter(deg_t, [iv], ones16)
            return 0
        lax.fori_loop(0, 4, cb, 0)

    issue(0, 0)
    issue(1, 1)

    def pair(p, _):
        g = 2 * p
        wait(0)
        consume(0)

        @pl.when(p < NGRPW // 2 - 1)
        def _():
            issue(0, g + 2)

        wait(1)
        consume(1)

        @pl.when(p < NGRPW // 2 - 1)
        def _():
            issue(1, g + 3)

        return 0
    lax.fori_loop(0, NGRPW // 2, pair, 0)

    pltpu.sync_copy(deg_t, degs.at[wid])


def _scd(dst2):
    return pl.kernel(
        _scd_body,
        out_type=[jax.ShapeDtypeStruct((NW, NP), f32)],
        mesh=_mesh,
        compiler_params=_sc_params,
        scratch_types=[
            pltpu.VMEM((4, 128), i32),
            pltpu.VMEM((4, 128), i32),
            pltpu.VMEM((NP,), f32),
            pltpu.SemaphoreType.DMA,
            pltpu.SemaphoreType.DMA,
        ],
    )(dst2)


def _tc1_body(d_ref, m_ref, wd_ref, wm_ref, degs_ref,
              feats_ref, x0h_ref, norm_ref):
    zd = jnp.dot(d_ref[...], wd_ref[...], preferred_element_type=f32)
    zm = jnp.dot(m_ref[...], wm_ref[...], preferred_element_type=f32)
    feats_ref[pl.ds(0, ND), :] = zd
    feats_ref[pl.ds(ND, ND), :] = zm
    feats_ref[pl.ds(N, NP - N), :] = jnp.zeros((NP - N, HID), f32)
    f = feats_ref[...]
    x0h_ref[0] = 0.5 * f[:, :FH]
    x0h_ref[1] = 0.5 * f[:, FH:]
    deg = jnp.clip(jnp.sum(degs_ref[...], axis=0), 1.0, None)
    norm_ref[...] = lax.rsqrt(deg)


def _tc1(d_sim, m_sim, W_d, W_m, degs):
    return pl.pallas_call(
        _tc1_body,
        out_shape=[
            jax.ShapeDtypeStruct((NP, HID), f32),
            jax.ShapeDtypeStruct((2, NP, FH), f32),
            jax.ShapeDtypeStruct((NP,), f32),
        ],
    )(d_sim, m_sim, W_d, W_m, degs)


def _tc2_body(yh_ref, wg_ref, al_ref, ar_ref, h2_ref, el_ref, er_ref, c_ref):
    X = jnp.concatenate([yh_ref[0], yh_ref[1]], axis=1)
    h = jnp.dot(X, wg_ref[...], preferred_element_type=f32)
    el = jnp.dot(h, al_ref[...], preferred_element_type=f32)
    er = jnp.dot(h, ar_ref[...], preferred_element_type=f32)
    h2_ref[0] = h[:, :FH]
    h2_ref[1] = h[:, FH:]
    el_ref[...] = el
    er_ref[...] = er
    t = jnp.max(el) + jnp.max(er)
    c = jnp.where(t > 0, t, SLOPE * t)
    c_ref[...] = jnp.full((128,), c, f32)


def _tc2(yh, W_gat, attn_l, attn_r):
    return pl.pallas_call(
        _tc2_body,
        out_shape=[
            jax.ShapeDtypeStruct((2, NP, FH), f32),
            jax.ShapeDtypeStruct((NP,), f32),
            jax.ShapeDtypeStruct((NP,), f32),
            jax.ShapeDtypeStruct((128,), f32),
        ],
    )(yh, W_gat, attn_l, attn_r)


def _tc3_body(numer_ref, dens_ref, feats_ref, wm1_ref, bm1_ref, wd1_ref,
              bd1_ref, wp_ref, bp_ref, s1_ref, s2_ref):
    den = jnp.clip(jnp.sum(dens_ref[...], axis=0), 1e-9, None)
    gat = jnp.concatenate([numer_ref[0], numer_ref[1]], axis=1) / den[:, None]
    m = jnp.max(gat, axis=-1, keepdims=True)
    feat0 = gat - (m + jnp.log(jnp.sum(jnp.exp(gat - m), axis=-1, keepdims=True)))
    f = feats_ref[...]
    wd1 = wd1_ref[...]
    wm1 = wm1_ref[...]
    a_d = (jnp.dot(feat0[:ND], wd1[:HID], preferred_element_type=f32)
           + jnp.dot(f[:ND], wd1[HID:], preferred_element_type=f32)
           + bd1_ref[...])
    a_m = (jnp.dot(feat0[ND:N], wm1[:HID], preferred_element_type=f32)
           + jnp.dot(f[ND:N], wm1[HID:], preferred_element_type=f32)
           + bm1_ref[...])
    h_d = jnp.where(a_d > 0, a_d, jnp.exp(a_d) - 1.0)
    h_m = jnp.where(a_m > 0, a_m, jnp.exp(a_m) - 1.0)
    wp1 = wp_ref[...][:HID, 0]
    wp2 = wp_ref[...][HID:, 0]
    bp = bp_ref[...]
    s1_ref[pl.ds(0, ND)] = jnp.dot(h_d, wp1, preferred_element_type=f32) + bp
    s1_ref[pl.ds(ND, ND)] = jnp.dot(h_m, wp1, preferred_element_type=f32) + bp
    s1_ref[pl.ds(N, NP - N)] = jnp.zeros((NP - N,), f32)
    s2_ref[pl.ds(0, ND)] = jnp.dot(h_d, wp2, preferred_element_type=f32)
    s2_ref[pl.ds(ND, ND)] = jnp.dot(h_m, wp2, preferred_element_type=f32)
    s2_ref[pl.ds(N, NP - N)] = jnp.zeros((NP - N,), f32)


def _tc3(numer, dens, featsP, Wm1, bm1, Wd1, bd1, Wp, bp):
    return pl.pallas_call(
        _tc3_body,
        out_shape=[
            jax.ShapeDtypeStruct((NP,), f32),
            jax.ShapeDtypeStruct((NP,), f32),
        ],
    )(numer, dens, featsP, Wm1, bm1, Wd1, bd1, Wp, bp)


def _sca_body(x0h, src2, dst2, normP, yh,
              u_sp, s_sp,
              sidxA, sidxB, didxA, didxB, gbufA, gbufB, zgbuf,
              xsl, ysl, normsv,
              isemA, isemB, gsemA, gsemB, ssemA, ssemB):
    cid = lax.axis_index("c")
    sid = lax.axis_index("s")
    row0 = sid * RPT
    grp0 = sid * (EPT // 128)

    z16 = jnp.zeros((L,), f32)

    def zero2d(ref, rows):
        def b(i, _):
            ref[i, pl.ds(0, L)] = z16
            ref[i, pl.ds(L, L)] = z16
            return 0
        lax.fori_loop(0, rows, b, 0)

    zero2d(ysl, RPT)
    zero2d(zgbuf, 128)
    pltpu.sync_copy(normP.at[pl.ds(row0, RPT)], normsv)

    def issue_idx(slot, g):
        sb, db, isem = ((sidxA, didxA, isemA) if slot == 0
                        else (sidxB, didxB, isemB))
        pltpu.async_copy(src2.at[pl.ds(grp0 + g * 4, 4)], sb, isem)
        pltpu.async_copy(dst2.at[pl.ds(grp0 + g * 4, 4)], db, isem)

    def wait_idx(slot):
        sb, db, isem = ((sidxA, didxA, isemA) if slot == 0
                        else (sidxB, didxB, isemB))
        pltpu.make_async_copy(src2.at[pl.ds(0, 4)], sb, isem).wait()
        pltpu.make_async_copy(dst2.at[pl.ds(0, 4)], db, isem).wait()

    def gathers(slot):
        sb, gb, gsem = ((sidxA, gbufA, gsemA) if slot == 0
                        else (sidxB, gbufB, gsemB))
        for j in range(4):
            pltpu.async_copy(u_sp.at[sb.at[j]],
                             gb.at[pl.ds(j * 128, 128)], gsem)

    def drain_gathers(slot):
        sb, gb, gsem = ((sidxA, gbufA, gsemA) if slot == 0
                        else (sidxB, gbufB, gsemB))
        for j in range(4):
            pltpu.make_async_copy(u_sp.at[sb.at[j]],
                                  gb.at[pl.ds(j * 128, 128)], gsem).wait()

    def scatters(slot):
        db, gb, ssem = ((didxA, gbufA, ssemA) if slot == 0
                        else (didxB, gbufB, ssemB))
        for j in range(4):
            pltpu.async_copy(gb.at[pl.ds(j * 128, 128)],
                             s_sp.at[db.at[j]], ssem, add=True)

    def drain_scatters(slot):
        db, gb, ssem = ((didxA, gbufA, ssemA) if slot == 0
                        else (didxB, gbufB, ssemB))
        for j in range(4):
            pltpu.make_async_copy(gb.at[pl.ds(j * 128, 128)],
                                  s_sp.at[db.at[j]], ssem).wait()

    def edge_pass():
        issue_idx(0, 0)
        issue_idx(1, 1)
        wait_idx(0)
        gathers(0)

        def pair(p, _):
            g = 2 * p
            wait_idx(1)
            drain_gathers(0)
            scatters(0)

            @pl.when(p < NGRP // 2 - 1)
            def _():
                issue_idx(0, g + 2)

            gathers(1)
            drain_gathers(1)
            scatters(1)

            @pl.when(p < NGRP // 2 - 1)
            def _():
                issue_idx(1, g + 3)

            drain_scatters(0)

            @pl.when(p < NGRP // 2 - 1)
            def _():
                wait_idx(0)
                gathers(0)

            drain_scatters(1)
            return 0
        lax.fori_loop(0, NGRP // 2, pair, 0)

    for t in range(K + 1):
        first = t == 0
        last = t == K
        if first:
            pltpu.sync_copy(x0h.at[cid, pl.ds(row0, RPT)], xsl)
        else:
            pltpu.sync_copy(s_sp.at[pl.ds(row0, RPT)], xsl)
        if not last:
            for zi in range(RPT // 128):
                pltpu.sync_copy(zgbuf, s_sp.at[pl.ds(row0 + zi * 128, 128)])

        def rowb(r, _, first=first, last=last):
            nv = plsc.load_gather(normsv, [jnp.full((L,), r, i32)])
            for half in range(2):
                v = xsl[r, pl.ds(half * L, L)]
                if not first:
                    v = v * nv
                yv = ysl[r, pl.ds(half * L, L)] + v
                ysl[r, pl.ds(half * L, L)] = yv
                if last:
                    xsl[r, pl.ds(half * L, L)] = yv * (1.0 / (K + 1))
                else:
                    xsl[r, pl.ds(half * L, L)] = v * nv
            return 0
        lax.fori_loop(0, RPT, rowb, 0)

        if last:
            pltpu.sync_copy(xsl, yh.at[cid, pl.ds(row0, RPT)])
        else:
            pltpu.sync_copy(xsl, u_sp.at[pl.ds(row0, RPT)])
            plsc.subcore_barrier()
            edge_pass()
            plsc.subcore_barrier()


def _sca(x0h, src2, dst2, normP):
    return pl.kernel(
        _sca_body,
        out_type=[jax.ShapeDtypeStruct((2, NP, FH), f32)],
        mesh=_mesh,
        compiler_params=_sc_params,
        scratch_types=[
            pltpu.VMEM_SHARED((NP, FH), f32),
            pltpu.VMEM_SHARED((NP, FH), f32),
            pltpu.VMEM((4, 128), i32),
            pltpu.VMEM((4, 128), i32),
            pltpu.VMEM((4, 128), i32),
            pltpu.VMEM((4, 128), i32),
            pltpu.VMEM((GEDG, FH), f32),
            pltpu.VMEM((GEDG, FH), f32),
            pltpu.VMEM((128, FH), f32),
            pltpu.VMEM((RPT, FH), f32),
            pltpu.VMEM((RPT, FH), f32),
            pltpu.VMEM((RPT,), f32),
            pltpu.SemaphoreType.DMA,
            pltpu.SemaphoreType.DMA,
            pltpu.SemaphoreType.DMA,
            pltpu.SemaphoreType.DMA,
            pltpu.SemaphoreType.DMA,
            pltpu.SemaphoreType.DMA,
        ],
    )(x0h, src2, dst2, normP)


def _scb_body(h2, el, er, crow, src2, dst2, numer, dens,
              h_sp, n_sp,
              sidxA, sidxB, didxA, didxB, gbufA, gbufB, zgbuf,
              elt, ert, dent, eebuf, cbuf,
              isemA, isemB, gsemA, gsemB, ssemA, ssemB):
    cid = lax.axis_index("c")
    sid = lax.axis_index("s")
    row0 = sid * RPT
    grp0 = sid * (EPT // 128)

    z16 = jnp.zeros((L,), f32)

    def zero1d(ref, n16):
        def b(i, _):
            ref[pl.ds(i * L, L)] = z16
            return 0
        lax.fori_loop(0, n16, b, 0)

    def zero2d(ref, rows):
        def b(i, _):
            ref[i, pl.ds(0, L)] = z16
            ref[i, pl.ds(L, L)] = z16
            return 0
        lax.fori_loop(0, rows, b, 0)

    zero1d(dent, NP // L)
    zero2d(zgbuf, 128)

    pltpu.sync_copy(h2.at[cid, pl.ds(row0, RPT)], h_sp.at[pl.ds(row0, RPT)])
    for zi in range(RPT // 128):
        pltpu.sync_copy(zgbuf, n_sp.at[pl.ds(row0 + zi * 128, 128)])
    pltpu.sync_copy(el, elt)
    pltpu.sync_copy(er, ert)
    pltpu.sync_copy(crow.at[pl.ds(0, L)], cbuf)
    cv = cbuf[pl.ds(0, L)]
    plsc.subcore_barrier()

    def issue_idx(slot, g):
        sb, db, isem = ((sidxA, didxA, isemA) if slot == 0
                        else (sidxB, didxB, isemB))
        pltpu.async_copy(src2.at[pl.ds(grp0 + g * 4, 4)], sb, isem)
        pltpu.async_copy(dst2.at[pl.ds(grp0 + g * 4, 4)], db, isem)

    def wait_idx(slot):
        sb, db, isem = ((sidxA, didxA, isemA) if slot == 0
                        else (sidxB, didxB, isemB))
        pltpu.make_async_copy(src2.at[pl.ds(0, 4)], sb, isem).wait()
        pltpu.make_async_copy(dst2.at[pl.ds(0, 4)], db, isem).wait()

    def gathers(slot):
        sb, gb, gsem = ((sidxA, gbufA, gsemA) if slot == 0
                        else (sidxB, gbufB, gsemB))
        for j in range(4):
            pltpu.async_copy(h_sp.at[sb.at[j]],
                             gb.at[pl.ds(j * 128, 128)], gsem)

    def drain_gathers(slot):
        sb, gb, gsem = ((sidxA, gbufA, gsemA) if slot == 0
                        else (sidxB, gbufB, gsemB))
        for j in range(4):
            pltpu.make_async_copy(h_sp.at[sb.at[j]],
                                  gb.at[pl.ds(j * 128, 128)], gsem).wait()

    def scatters(slot):
        db, gb, ssem = ((didxA, gbufA, ssemA) if slot == 0
                        else (didxB, gbufB, ssemB))
        for j in range(4):
            pltpu.async_copy(gb.at[pl.ds(j * 128, 128)],
                             n_sp.at[db.at[j]], ssem, add=True)

    def drain_scatters(slot):
        db, gb, ssem = ((didxA, gbufA, ssemA) if slot == 0
                        else (didxB, gbufB, ssemB))
        for j in range(4):
            pltpu.make_async_copy(gb.at[pl.ds(j * 128, 128)],
                                  n_sp.at[db.at[j]], ssem).wait()

    def edge_compute(slot):
        sb, db, gb = ((sidxA, didxA, gbufA) if slot == 0
                      else (sidxB, didxB, gbufB))

        def cb(q, _):
            for jj in range(8):
                svv = sb[q, pl.ds(jj * L, L)]
                dvv = db[q, pl.ds(jj * L, L)]
                ev = (plsc.load_gather(elt, [svv])
                      + plsc.load_gather(ert, [dvv]))
                ev = jnp.where(ev > 0, ev, SLOPE * ev)
                eev = jnp.exp(ev - cv)
                eebuf[pl.ds(q * 128 + jj * L, L)] = eev
                plsc.addupdate_scatter(dent, [dvv], eev)
            return 0
        lax.fori_loop(0, 4, cb, 0)

        def rb(rr, _):
            evs = [plsc.load_gather(eebuf, [jnp.full((L,), rr * 16 + u, i32)])
                   for u in range(16)]
            for u in range(16):
                r = rr * 16 + u
                gb[r, pl.ds(0, L)] = gb[r, pl.ds(0, L)] * evs[u]
                gb[r, pl.ds(L, L)] = gb[r, pl.ds(L, L)] * evs[u]
            return 0
        lax.fori_loop(0, GEDG // 16, rb, 0)

    issue_idx(0, 0)
    issue_idx(1, 1)
    wait_idx(0)
    gathers(0)

    def pair(p, _):
        g = 2 * p
        wait_idx(1)
        drain_gathers(0)
        edge_compute(0)
        scatters(0)

        @pl.when(p < NGRP // 2 - 1)
        def _():
            issue_idx(0, g + 2)

        gathers(1)
        drain_gathers(1)
        edge_compute(1)
        scatters(1)

        @pl.when(p < NGRP // 2 - 1)
        def _():
            issue_idx(1, g + 3)

        drain_scatters(0)

        @pl.when(p < NGRP // 2 - 1)
        def _():
            wait_idx(0)
            gathers(0)

        drain_scatters(1)
        return 0
    lax.fori_loop(0, NGRP // 2, pair, 0)

    plsc.subcore_barrier()
    pltpu.sync_copy(n_sp.at[pl.ds(row0, RPT)], numer.at[cid, pl.ds(row0, RPT)])
    pltpu.sync_copy(dent, dens.at[sid])


def _scb(h2, el, er, crow, src2, dst2):
    return pl.kernel(
        _scb_body,
        out_type=[
            jax.ShapeDtypeStruct((2, NP, FH), f32),
            jax.ShapeDtypeStruct((NS, NP), f32),
        ],
        mesh=_mesh,
        compiler_params=_sc_params,
        scratch_types=[
            pltpu.VMEM_SHARED((NP, FH), f32),
            pltpu.VMEM_SHARED((NP, FH), f32),
            pltpu.VMEM((4, 128), i32),
            pltpu.VMEM((4, 128), i32),
            pltpu.VMEM((4, 128), i32),
            pltpu.VMEM((4, 128), i32),
            pltpu.VMEM((GEDG, FH), f32),
            pltpu.VMEM((GEDG, FH), f32),
            pltpu.VMEM((128, FH), f32),
            pltpu.VMEM((NP,), f32),
            pltpu.VMEM((NP,), f32),
            pltpu.VMEM((NP,), f32),
            pltpu.VMEM((GEDG,), f32),
            pltpu.VMEM((L,), f32),
            pltpu.SemaphoreType.DMA,
            pltpu.SemaphoreType.DMA,
            pltpu.SemaphoreType.DMA,
            pltpu.SemaphoreType.DMA,
            pltpu.SemaphoreType.DMA,
            pltpu.SemaphoreType.DMA,
        ],
    )(h2, el, er, crow, src2, dst2)


def _scc_body(s1h, s2h, dis, mir, out, s1t, s2t, dxt, mxt, obuf):
    cid = lax.axis_index("c")
    sid = lax.axis_index("s")
    wid = sid * NC + cid
    npt = BPAIR // NW
    base = wid * npt
    pltpu.sync_copy(s1h, s1t)
    pltpu.sync_copy(s2h, s2t)
    pltpu.sync_copy(dis.at[pl.ds(base, npt)], dxt)
    pltpu.sync_copy(mir.at[pl.ds(base, npt)], mxt)

    def b(j, _):
        iv = dxt[pl.ds(j * L, L)]
        jv = mxt[pl.ds(j * L, L)]
        a = plsc.load_gather(s1t, [iv]) + plsc.load_gather(s2t, [jv])
        obuf[pl.ds(j * L, L)] = 1.0 / (1.0 + jnp.exp(-a))
        return 0
    lax.fori_loop(0, npt // L, b, 0)
    pltpu.sync_copy(obuf, out.at[pl.ds(base, npt)])


def _scc(s1P, s2P, diseases, mirnas):
    npt = BPAIR // NW
    return pl.kernel(
        _scc_body,
        out_type=[jax.ShapeDtypeStruct((BPAIR,), f32)],
        mesh=_mesh,
        compiler_params=_sc_params,
        scratch_types=[
            pltpu.VMEM((NP,), f32),
            pltpu.VMEM((NP,), f32),
            pltpu.VMEM((npt,), i32),
            pltpu.VMEM((npt,), i32),
            pltpu.VMEM((npt,), f32),
        ],
    )(s1P, s2P, diseases, mirnas)


def kernel(d_sim, m_sim, W_d, W_m, W_gat, attn_l, attn_r, Wm1, bm1, Wd1, bd1,
           Wp, bp, edge_index, diseases, mirnas):
    pad = jnp.full((EP - E,), PADIDX, i32)
    src2 = jnp.concatenate([edge_index[0].astype(i32), pad]).reshape(ROWS8, 128)
    dst2 = jnp.concatenate([edge_index[1].astype(i32), pad]).reshape(ROWS8, 128)

    (degs,) = _scd(dst2)
    featsP, x0h, normP = _tc1(d_sim, m_sim, W_d, W_m, degs)
    (yh,) = _sca(x0h, src2, dst2, normP)
    h2, el, er, crow = _tc2(yh, W_gat, attn_l, attn_r)
    numer, dens = _scb(h2, el, er, crow, src2, dst2)
    s1P, s2P = _tc3(numer, dens, featsP, Wm1, bm1, Wd1, bd1, Wp, bp)
    (score,) = _scc(s1P, s2P, diseases.astype(i32), mirnas.astype(i32))
    return score.reshape(BPAIR, 1)
